# baseline (speedup 1.0000x reference)
import functools

import jax
import jax.numpy as jnp
from jax import lax
from jax.experimental import pallas as pl
from jax.experimental.pallas import tpu as pltpu
from jax.experimental.pallas import tpu_sc as plsc

VOCAB = 1000000
N_EMBD = 64
BATCH = 16384
HIST = 50

NC = 2
NS = 16
NW = NC * NS
CB = 128
UNITS = HIST * (BATCH // CB)
UPW = UNITS // NW
NGB = 4
NOB = 2
NIB = 4
F_G = 2
F_I = 3
UNROLL = 4

assert UNITS % NW == 0 and UPW % UNROLL == 0
assert UNROLL % NGB == 0 and UNROLL % NOB == 0 and UNROLL % NIB == 0

_mesh = plsc.VectorSubcoreMesh(core_axis_name="c", subcore_axis_name="s")


@functools.partial(
    pl.kernel,
    out_type=jax.ShapeDtypeStruct((HIST, N_EMBD, BATCH), jnp.float32),
    mesh=_mesh,
    compiler_params=pltpu.CompilerParams(
        use_tc_tiling_on_sc=True, needs_layout_passes=False,
        disable_bounds_checks=True, disable_semaphore_checks=True),
    scratch_types=[
        pltpu.VMEM((NIB, CB), jnp.int32),
        pltpu.VMEM((NGB, CB, 128), jnp.float32),
        pltpu.VMEM((CB * 73,), jnp.float32),
        pltpu.VMEM((NOB, N_EMBD, CB), jnp.float32),
        pltpu.SemaphoreType.DMA,
        pltpu.SemaphoreType.DMA,
        pltpu.SemaphoreType.DMA,
    ],
)
def _embed_lookup(xt_hbm, tbl_hbm, out_hbm, idx_v, gbuf, tmp,
                  oblk, isem, gsem, osem):
    wid = lax.axis_index("s") * NC + lax.axis_index("c")
    u0 = wid * UPW

    def fire_idx(u, s):
        h = u // (BATCH // CB)
        c = u % (BATCH // CB)
        pltpu.async_copy(xt_hbm.at[h, pl.ds(c * CB, CB)], idx_v.at[s], isem)

    def wait_idx(s):
        pltpu.make_async_copy(xt_hbm.at[0, pl.ds(0, CB)], idx_v.at[s], isem).wait()

    def fire_gather(si, sg):
        pltpu.async_copy(tbl_hbm.at[idx_v.at[si]], gbuf.at[sg], gsem)

    def drain_gather(sg):
        pltpu.make_async_copy(tbl_hbm.at[idx_v.at[0]], gbuf.at[sg], gsem).wait()

    def build_block(sg, so):
        for j in range(CB):
            for k in range(4):
                tmp[pl.ds(j * 73 + k * 16, 16)] = (
                    gbuf[sg, j, pl.ds(k * 16, 16)])

        bases = [(lax.iota(jnp.int32, 16) + 16 * k) * 73
                 for k in range(CB // 16)]

        for d in range(N_EMBD):
            vals = [plsc.load_gather(tmp, [bases[k] + d])
                    for k in range(CB // 16)]
            for k in range(CB // 16):
                oblk[so, d, pl.ds(k * 16, 16)] = vals[k]

    def put_block(u, so):
        h = u // (BATCH // CB)
        c = u % (BATCH // CB)
        pltpu.async_copy(
            oblk.at[so], out_hbm.at[h, :, pl.ds(c * CB, CB)], osem)

    def wait_put(so):
        pltpu.make_async_copy(
            oblk.at[so], out_hbm.at[0, :, pl.ds(0, CB)], osem).wait()

    for k in range(F_I):
        fire_idx(u0 + k, k)
    for k in range(F_G):
        wait_idx(k)
        fire_gather(k, k)

    def body(t, _):
        for b in range(UNROLL):
            u_local = t * UNROLL + b
            u = u0 + u_local
            sg = b % NGB
            so = b % NOB

            @pl.when(u_local + F_I < UPW)
            def _():
                fire_idx(u + F_I, (b + F_I) % NIB)

            @pl.when(u_local + F_G < UPW)
            def _():
                wait_idx((b + F_G) % NIB)
                fire_gather((b + F_G) % NIB, (b + F_G) % NGB)

            drain_gather(sg)

            @pl.when(u_local >= NOB)
            def _():
                wait_put(so)

            build_block(sg, so)
            put_block(u, so)
        return 0

    lax.fori_loop(0, UPW // UNROLL, body, 0)
    for so in range(NOB):
        wait_put(so)


def kernel(x, table):
    xt = jnp.transpose(x.astype(jnp.int32))
    tbl = jnp.pad(table, ((0, 0), (0, N_EMBD)))
    out_t = _embed_lookup(xt, tbl)
    return jnp.transpose(out_t, (2, 0, 1))


# speedup vs baseline: 1.4634x; 1.4634x over previous
import functools

import jax
import jax.numpy as jnp
from jax import lax
from jax.experimental import pallas as pl
from jax.experimental.pallas import tpu as pltpu
from jax.experimental.pallas import tpu_sc as plsc

VOCAB = 1000000
N_EMBD = 64
BATCH = 16384
HIST = 50

NC = 2
NS = 16
NW = NC * NS
CB = 128
UNITS = HIST * (BATCH // CB)
UPW = UNITS // NW
NGB = 4
NOB = 2
NIB = 4
F_G = 2
F_I = 3
UNROLL = 4

assert UNITS % NW == 0 and UPW % UNROLL == 0
assert UNROLL % NGB == 0 and UNROLL % NOB == 0 and UNROLL % NIB == 0

_mesh = plsc.VectorSubcoreMesh(core_axis_name="c", subcore_axis_name="s")


@functools.partial(
    pl.kernel,
    out_type=jax.ShapeDtypeStruct((HIST, N_EMBD, BATCH), jnp.float32),
    mesh=_mesh,
    compiler_params=pltpu.CompilerParams(
        use_tc_tiling_on_sc=True, needs_layout_passes=False,
        disable_bounds_checks=True, disable_semaphore_checks=True),
    scratch_types=[
        pltpu.VMEM((NIB, CB), jnp.int32),
        pltpu.VMEM((NGB, CB, 128), jnp.float32),
        pltpu.VMEM((CB * 73,), jnp.float32),
        pltpu.VMEM((NOB, N_EMBD, CB), jnp.float32),
        pltpu.SemaphoreType.DMA,
        pltpu.SemaphoreType.DMA,
        pltpu.SemaphoreType.DMA,
    ],
)
def _embed_lookup(xt_hbm, tbl_hbm, out_hbm, idx_v, gbuf, tmp,
                  oblk, isem, gsem, osem):
    wid = lax.axis_index("s") * NC + lax.axis_index("c")
    u0 = wid * UPW

    def fire_idx(u, s):
        h = u // (BATCH // CB)
        c = u % (BATCH // CB)
        pltpu.async_copy(xt_hbm.at[h, pl.ds(c * CB, CB)], idx_v.at[s], isem)

    def wait_idx(s):
        pltpu.make_async_copy(xt_hbm.at[0, pl.ds(0, CB)], idx_v.at[s], isem).wait()

    def fire_gather(si, sg):
        pltpu.async_copy(tbl_hbm.at[idx_v.at[si]], gbuf.at[sg], gsem)

    def drain_gather(sg):
        pltpu.make_async_copy(tbl_hbm.at[idx_v.at[0]], gbuf.at[sg], gsem).wait()

    def build_block(sg, so):
        for j in range(CB):
            for k in range(4):
                tmp[pl.ds(j * 73 + k * 16, 16)] = (
                    gbuf[sg, j, pl.ds(k * 16, 16)])

        bases = [(lax.iota(jnp.int32, 16) + 16 * k) * 73
                 for k in range(CB // 16)]

        def dstep(d4, _):
            for dd in range(4):
                d = d4 * 4 + dd
                vals = [plsc.load_gather(tmp, [bases[k] + d])
                        for k in range(CB // 16)]
                for k in range(CB // 16):
                    oblk[so, d, pl.ds(k * 16, 16)] = vals[k]
            return 0

        lax.fori_loop(0, N_EMBD // 4, dstep, 0)

    def put_block(u, so):
        h = u // (BATCH // CB)
        c = u % (BATCH // CB)
        pltpu.async_copy(
            oblk.at[so], out_hbm.at[h, :, pl.ds(c * CB, CB)], osem)

    def wait_put(so):
        pltpu.make_async_copy(
            oblk.at[so], out_hbm.at[0, :, pl.ds(0, CB)], osem).wait()

    for k in range(F_I):
        fire_idx(u0 + k, k)
    for k in range(F_G):
        wait_idx(k)
        fire_gather(k, k)

    def body(t, _):
        for b in range(UNROLL):
            u_local = t * UNROLL + b
            u = u0 + u_local
            sg = b % NGB
            so = b % NOB

            @pl.when(u_local + F_I < UPW)
            def _():
                fire_idx(u + F_I, (b + F_I) % NIB)

            @pl.when(u_local + F_G < UPW)
            def _():
                wait_idx((b + F_G) % NIB)
                fire_gather((b + F_G) % NIB, (b + F_G) % NGB)

            drain_gather(sg)

            @pl.when(u_local >= NOB)
            def _():
                wait_put(so)

            build_block(sg, so)
            put_block(u, so)
        return 0

    lax.fori_loop(0, UPW // UNROLL, body, 0)
    for so in range(NOB):
        wait_put(so)


def kernel(x, table):
    xt = jnp.transpose(x.astype(jnp.int32))
    tbl = jnp.pad(table, ((0, 0), (0, N_EMBD)))
    out_t = _embed_lookup(xt, tbl)
    return jnp.transpose(out_t, (2, 0, 1))
